# baseline (speedup 1.0000x reference)
import functools

import jax
import jax.numpy as jnp
import numpy as np
from jax.experimental import pallas as pl
from jax.experimental.pallas import tpu as pltpu

N = 10000
E = 320000
NG = 16
RATIO = 0.8


def _gcn(x, W, b, src, dst, emask):
    xw = x @ W
    ew = emask.astype(x.dtype)
    deg = jnp.zeros((x.shape[0],), x.dtype).at[dst].add(ew) + 1.0
    dinv = deg ** -0.5
    coef = dinv[src] * dinv[dst] * ew
    out = jnp.zeros_like(xw).at[dst].add(coef[:, None] * xw[src])
    out = out + (dinv * dinv)[:, None] * xw
    return out + b


def _pool(x, p, src, dst, emask, nmask, batch):
    score = jnp.tanh((x @ p) / jnp.linalg.norm(p))
    eff = jnp.where(nmask, score, -jnp.inf)
    order = jnp.lexsort((-eff, batch))
    totals = jax.ops.segment_sum(jnp.ones_like(batch), batch, num_segments=NG)
    starts = jnp.cumsum(totals) - totals
    ranks = jnp.arange(x.shape[0]) - starts[batch[order]]
    node_rank = jnp.zeros((x.shape[0],), dtype=ranks.dtype).at[order].set(ranks)
    alive = jax.ops.segment_sum(nmask.astype(jnp.int32), batch, num_segments=NG)
    k = jnp.ceil(RATIO * alive.astype(jnp.float32)).astype(jnp.int32)
    new_mask = (node_rank < k[batch]) & nmask
    xg = x * score[:, None] * new_mask[:, None].astype(x.dtype)
    new_emask = emask & new_mask[src] & new_mask[dst]
    return xg, new_mask, new_emask


def _readout(x, batch, nmask):
    xm = jnp.where(nmask[:, None], x, -jnp.inf)
    mx = jax.ops.segment_max(xm, batch, num_segments=NG)
    s = jax.ops.segment_sum(x, batch, num_segments=NG)
    c = jax.ops.segment_sum(nmask.astype(x.dtype), batch, num_segments=NG)
    return jnp.concatenate([mx, s / c[:, None]], axis=1)


def _block(h, W, b, p, src, dst, emask, nmask, batch):
    h = jax.nn.relu(_gcn(h, W, b, src, dst, emask))
    h, nmask, emask = _pool(h, p, src, dst, emask, nmask, batch)
    r = _readout(h, batch, nmask)
    return h, nmask, emask, r


def _head_kernel(z_ref, lw1_ref, lb1_ref, lw2_ref, lb2_ref, lw3_ref, lb3_ref, o_ref):
    z = z_ref[...]
    z = jax.nn.relu(z @ lw1_ref[...] + lb1_ref[...])
    z = jax.nn.relu(z @ lw2_ref[...] + lb2_ref[...])
    logits = z @ lw3_ref[...] + lb3_ref[...]
    o_ref[...] = jax.nn.log_softmax(logits, axis=-1)


def kernel(x, edge_index, batch, W1, b1, p1, W2, b2, p2, W3, b3, p3, W4, b4, p4, lw1, lb1, lw2, lb2, lw3, lb3):
    src = edge_index[0]
    dst = edge_index[1]
    nmask = jnp.ones((x.shape[0],), dtype=jnp.bool_)
    emask = jnp.ones((src.shape[0],), dtype=jnp.bool_)
    h = x
    h, nmask, emask, r1 = _block(h, W1, b1, p1, src, dst, emask, nmask, batch)
    h, nmask, emask, r2 = _block(h, W2, b2, p2, src, dst, emask, nmask, batch)
    h, nmask, emask, r3 = _block(h, W3, b3, p3, src, dst, emask, nmask, batch)
    h, nmask, emask, r4 = _block(h, W4, b4, p4, src, dst, emask, nmask, batch)
    z = r1 + r2 + r3 + r4
    out = pl.pallas_call(
        _head_kernel,
        out_shape=jax.ShapeDtypeStruct((NG, lw3.shape[1]), jnp.float32),
    )(z, lw1, lb1[None, :], lw2, lb2[None, :], lw3, lb3[None, :])
    return out


# speedup vs baseline: 1.0695x; 1.0695x over previous
import jax
import jax.numpy as jnp
from jax.experimental import pallas as pl

N = 10000
E = 320000
NG = 16
RATIO = 0.8


def _k1_matmul(h_ref, w_ref, o_ref):
    o_ref[...] = h_ref[...] @ w_ref[...]


def _k2_combine_score(agg_ref, xw_ref, dinv2_ref, b_ref, p_ref, o_ref, s_ref):
    out = agg_ref[...] + dinv2_ref[...] * xw_ref[...] + b_ref[...]
    out = jnp.maximum(out, 0.0)
    p = p_ref[...]
    pn = jax.lax.rsqrt(jnp.sum(p * p))
    o_ref[...] = out
    s_ref[...] = jnp.tanh((out @ p) * pn)


def _k3_gate_readout(x_ref, score_ref, mask_ref, batch_col_ref, batch_row_ref,
                     xg_ref, r_ref):
    x = x_ref[...]
    mask = mask_ref[...]
    xg = x * score_ref[...] * mask
    xg_ref[...] = xg
    gids = jax.lax.broadcasted_iota(jnp.int32, (NG, N), 0)
    seg = (batch_row_ref[...] == gids).astype(jnp.float32)
    s = seg @ xg
    c = seg @ mask
    r_ref[:, 128:] = s / c
    bcol = batch_col_ref[...]
    neg = jnp.full_like(xg, -jnp.inf)
    alive = mask > 0.0
    for g in range(NG):
        rows = jnp.where(alive & (bcol == g), xg, neg)
        r_ref[g, 0:128] = jnp.max(rows, axis=0)


def _k4_head(z_ref, lw1_ref, lb1_ref, lw2_ref, lb2_ref, lw3_ref, lb3_ref, o_ref):
    z = z_ref[...]
    z = jnp.maximum(z @ lw1_ref[...] + lb1_ref[...], 0.0)
    z = jnp.maximum(z @ lw2_ref[...] + lb2_ref[...], 0.0)
    logits = z @ lw3_ref[...] + lb3_ref[...]
    o_ref[...] = jax.nn.log_softmax(logits, axis=-1)


def _block(h, W, b, p, src, dst, emask, nmask, batch, batch_col, batch_row):
    f32 = jnp.float32
    xw = pl.pallas_call(
        _k1_matmul,
        out_shape=jax.ShapeDtypeStruct((N, 128), f32),
    )(h, W)

    ew = emask.astype(f32)
    deg = jnp.zeros((N,), f32).at[dst].add(ew) + 1.0
    dinv = deg ** -0.5
    coef = dinv[src] * dinv[dst] * ew
    agg = jnp.zeros((N, 128), f32).at[dst].add(coef[:, None] * xw[src])

    out, score = pl.pallas_call(
        _k2_combine_score,
        out_shape=(
            jax.ShapeDtypeStruct((N, 128), f32),
            jax.ShapeDtypeStruct((N, 1), f32),
        ),
    )(agg, xw, (dinv * dinv)[:, None], b[None, :], p[:, None])

    score1 = score[:, 0]
    eff = jnp.where(nmask, score1, -jnp.inf)
    order = jnp.lexsort((-eff, batch))
    totals = jax.ops.segment_sum(jnp.ones_like(batch), batch, num_segments=NG)
    starts = jnp.cumsum(totals) - totals
    ranks = jnp.arange(N) - starts[batch[order]]
    node_rank = jnp.zeros((N,), dtype=ranks.dtype).at[order].set(ranks)
    alive = jax.ops.segment_sum(nmask.astype(jnp.int32), batch, num_segments=NG)
    k = jnp.ceil(RATIO * alive.astype(f32)).astype(jnp.int32)
    new_mask = (node_rank < k[batch]) & nmask

    xg, r = pl.pallas_call(
        _k3_gate_readout,
        out_shape=(
            jax.ShapeDtypeStruct((N, 128), f32),
            jax.ShapeDtypeStruct((NG, 256), f32),
        ),
    )(out, score, new_mask.astype(f32)[:, None], batch_col, batch_row)

    new_emask = emask & new_mask[src] & new_mask[dst]
    return xg, new_mask, new_emask, r


def kernel(x, edge_index, batch, W1, b1, p1, W2, b2, p2, W3, b3, p3,
           W4, b4, p4, lw1, lb1, lw2, lb2, lw3, lb3):
    src = edge_index[0]
    dst = edge_index[1]
    batch_col = batch[:, None]
    batch_row = batch[None, :]
    nmask = jnp.ones((N,), dtype=jnp.bool_)
    emask = jnp.ones((E,), dtype=jnp.bool_)
    h = x
    h, nmask, emask, r1 = _block(h, W1, b1, p1, src, dst, emask, nmask, batch, batch_col, batch_row)
    h, nmask, emask, r2 = _block(h, W2, b2, p2, src, dst, emask, nmask, batch, batch_col, batch_row)
    h, nmask, emask, r3 = _block(h, W3, b3, p3, src, dst, emask, nmask, batch, batch_col, batch_row)
    h, nmask, emask, r4 = _block(h, W4, b4, p4, src, dst, emask, nmask, batch, batch_col, batch_row)
    z = r1 + r2 + r3 + r4
    return pl.pallas_call(
        _k4_head,
        out_shape=jax.ShapeDtypeStruct((NG, lw3.shape[1]), jnp.float32),
    )(z, lw1, lb1[None, :], lw2, lb2[None, :], lw3, lb3[None, :])
